# Initial kernel scaffold; baseline (speedup 1.0000x reference)
import functools

import jax
import jax.numpy as jnp
from jax import lax
from jax.experimental import pallas as pl
from jax.experimental.pallas import tpu as pltpu
from jax.experimental.pallas import tpu_sc as plsc

N_NODES = 10000
N_EDGES = 320000
NODE_DIM = 128
EDGE_DIM = 16
HIDDEN = 200
HP = 208
NSL = HP // 16
DEPTH = 4
NUM_GRAPHS = 64
EPS = 1e-5

NC = 2
NSUB = 16
NW = NC * NSUB
E_PER_W = N_EDGES // NW
CHUNK = 80
NCHUNK = E_PER_W // CHUNK
NROWS_W = N_NODES // NSUB

_mesh = plsc.VectorSubcoreMesh(core_axis_name="c", subcore_axis_name="s")


def _make_sc_scatter(use_norm):
    @functools.partial(
        pl.kernel,
        mesh=_mesh,
        out_type=jax.ShapeDtypeStruct((NC, N_NODES, HP), jnp.float32),
        scratch_types=[
            pltpu.VMEM_SHARED((N_NODES, HP), jnp.float32),
            pltpu.VMEM((CHUNK, HP), jnp.float32),
            pltpu.VMEM((CHUNK,), jnp.int32),
            pltpu.VMEM((HP,), jnp.float32),
            pltpu.VMEM((HP,), jnp.float32),
        ],
    )
    def k(z_hbm, col_hbm, zeros_hbm, isc_hbm, ish_hbm, out_hbm,
          acc_sh, z_v, idx_v, isc_v, ish_v):
        cid = lax.axis_index("c")
        sid = lax.axis_index("s")
        wid = cid * NSUB + sid
        pltpu.sync_copy(zeros_hbm.at[pl.ds(sid * NROWS_W, NROWS_W)],
                        acc_sh.at[pl.ds(sid * NROWS_W, NROWS_W)])
        plsc.subcore_barrier()
        pltpu.sync_copy(isc_hbm, isc_v)
        pltpu.sync_copy(ish_hbm, ish_v)
        iscs = [isc_v[pl.ds(s * 16, 16)] for s in range(NSL)]
        ishs = [ish_v[pl.ds(s * 16, 16)] for s in range(NSL)]
        base = wid * E_PER_W

        def chunk_body(i, carry):
            off = base + i * CHUNK
            pltpu.sync_copy(z_hbm.at[pl.ds(off, CHUNK)], z_v)
            pltpu.sync_copy(col_hbm.at[pl.ds(off, CHUNK)], idx_v)

            def row_body(r, c2):
                for s in range(NSL):
                    sl = pl.ds(s * 16, 16)
                    z = z_v[r, sl]
                    if use_norm:
                        t = z * iscs[s] + ishs[s]
                        h = jnp.maximum(t, 0.0)
                        h = h + h
                    else:
                        h = jnp.maximum(z, 0.01 * z)
                    z_v[r, sl] = h
                return c2

            lax.fori_loop(0, CHUNK, row_body, 0)
            pltpu.sync_copy(z_v, acc_sh.at[idx_v], add=True)
            return carry

        lax.fori_loop(0, NCHUNK, chunk_body, 0)
        plsc.subcore_barrier()
        pltpu.sync_copy(acc_sh.at[pl.ds(sid * NROWS_W, NROWS_W)],
                        out_hbm.at[cid, pl.ds(sid * NROWS_W, NROWS_W)])

    return k


def _make_sc_gather(flip, do_stats):
    out_type = [jax.ShapeDtypeStruct((N_EDGES, HP), jnp.float32)]
    if do_stats:
        out_type.append(jax.ShapeDtypeStruct((NW, 2, HP), jnp.float32))

    @functools.partial(
        pl.kernel,
        mesh=_mesh,
        out_type=out_type,
        scratch_types=[
            pltpu.VMEM((CHUNK, HP), jnp.float32),
            pltpu.VMEM((CHUNK, HP), jnp.float32),
            pltpu.VMEM((CHUNK,), jnp.int32),
            pltpu.VMEM((2, HP), jnp.float32),
            pltpu.SemaphoreType.DMA,
        ],
    )
    def k(table_hbm, q_hbm, idx_hbm, *refs):
        if do_stats:
            out_hbm, stats_hbm, q_v, g_v, idx_v, st_v, sem = refs
        else:
            out_hbm, q_v, g_v, idx_v, st_v, sem = refs
        cid = lax.axis_index("c")
        sid = lax.axis_index("s")
        wid = cid * NSUB + sid
        base = wid * E_PER_W
        zero = jnp.zeros((16,), jnp.float32)
        init = tuple([zero] * NSL) + tuple([zero] * NSL)

        def chunk_body(i, carry):
            off = base + i * CHUNK
            pltpu.sync_copy(idx_hbm.at[pl.ds(off, CHUNK)], idx_v)
            pltpu.sync_copy(q_hbm.at[pl.ds(off, CHUNK)], q_v)
            pltpu.async_copy(table_hbm.at[idx_v], g_v, sem).wait()

            def pair_body(r, c2):
                sums = c2[:NSL]
                sqs = c2[NSL:]
                new_s = []
                new_q = []
                for s in range(NSL):
                    sl = pl.ds(s * 16, 16)
                    qa = q_v[2 * r, sl]
                    qb = q_v[2 * r + 1, sl]
                    ga = g_v[2 * r, sl]
                    gb = g_v[2 * r + 1, sl]
                    if flip:
                        y0 = ga - qb
                        y1 = gb - qa
                    else:
                        y0 = ga + qa
                        y1 = gb + qb
                    q_v[2 * r, sl] = y0
                    q_v[2 * r + 1, sl] = y1
                    if do_stats:
                        new_s.append(sums[s] + (y0 + y1))
                        new_q.append(sqs[s] + (y0 * y0 + y1 * y1))
                if do_stats:
                    return tuple(new_s) + tuple(new_q)
                return c2

            carry = lax.fori_loop(0, CHUNK // 2, pair_body, carry)
            pltpu.sync_copy(q_v, out_hbm.at[pl.ds(off, CHUNK)])
            return carry

        fin = lax.fori_loop(0, NCHUNK, chunk_body, init)
        if do_stats:
            for s in range(NSL):
                st_v[0, pl.ds(s * 16, 16)] = fin[s]
                st_v[1, pl.ds(s * 16, 16)] = fin[NSL + s]
            pltpu.sync_copy(st_v, stats_hbm.at[wid])

    return k


_sc_scatter_init = _make_sc_scatter(use_norm=False)
_sc_scatter_norm = _make_sc_scatter(use_norm=True)
_sc_gather_init = _make_sc_gather(flip=False, do_stats=False)
_sc_gather_conv = _make_sc_gather(flip=True, do_stats=True)


def _mm_kernel(x_ref, w_ref, o_ref):
    o_ref[...] = jnp.dot(x_ref[...], w_ref[...],
                         preferred_element_type=jnp.float32)


def _tc_matmul(x, w, row_block):
    m, kdim = x.shape
    n = w.shape[1]
    return pl.pallas_call(
        _mm_kernel,
        grid=(m // row_block,),
        in_specs=[
            pl.BlockSpec((row_block, kdim), lambda i: (i, 0)),
            pl.BlockSpec((kdim, n), lambda i: (0, 0)),
        ],
        out_specs=pl.BlockSpec((row_block, n), lambda i: (i, 0)),
        out_shape=jax.ShapeDtypeStruct((m, n), jnp.float32),
    )(x, w)


def _make_tc_act_matmul(use_norm, row_block):
    def body(z_ref, w_ref, isc_ref, ish_ref, o_ref):
        z = z_ref[...]
        if use_norm:
            t = z * isc_ref[...] + ish_ref[...]
            h = jnp.maximum(t, 0.0)
            h = h + h
        else:
            h = jnp.maximum(z, 0.01 * z)
        o_ref[...] = jnp.dot(h, w_ref[...], preferred_element_type=jnp.float32)

    def call(z, w, isc, ish):
        m, kdim = z.shape
        n = w.shape[1]
        return pl.pallas_call(
            body,
            grid=(m // row_block,),
            in_specs=[
                pl.BlockSpec((row_block, kdim), lambda i: (i, 0)),
                pl.BlockSpec((kdim, n), lambda i: (0, 0)),
                pl.BlockSpec((1, kdim), lambda i: (0, 0)),
                pl.BlockSpec((1, kdim), lambda i: (0, 0)),
            ],
            out_specs=pl.BlockSpec((row_block, n), lambda i: (i, 0)),
            out_shape=jax.ShapeDtypeStruct((m, n), jnp.float32),
        )(z, w, isc, ish)

    return call


_tc_q_init = _make_tc_act_matmul(use_norm=False, row_block=640)
_tc_q_norm = _make_tc_act_matmul(use_norm=True, row_block=640)


def _tc_pprime(A, w, b):
    def body(a_ref, w_ref, b_ref, o_ref):
        a = a_ref[0] + a_ref[1]
        o_ref[...] = jnp.dot(a, w_ref[...],
                             preferred_element_type=jnp.float32) + b_ref[...]

    return pl.pallas_call(
        body,
        out_shape=jax.ShapeDtypeStruct((N_NODES, HP), jnp.float32),
    )(A, w, b)


def _tc_stats(stats, gcp, bep):
    def body(st_ref, g_ref, be_ref, isc_ref, ish_ref):
        s = jnp.sum(st_ref[:, 0, :], axis=0, keepdims=True)
        sq = jnp.sum(st_ref[:, 1, :], axis=0, keepdims=True)
        mu = s / N_EDGES
        var = sq / N_EDGES - mu * mu
        isc = g_ref[...] * jax.lax.rsqrt(var + EPS)
        isc_ref[...] = isc
        ish_ref[...] = be_ref[...] - mu * isc

    return pl.pallas_call(
        body,
        out_shape=[
            jax.ShapeDtypeStruct((1, HP), jnp.float32),
            jax.ShapeDtypeStruct((1, HP), jnp.float32),
        ],
    )(stats, gcp, bep)


def _tc_final(A, wf, batch2d):
    def body(a_ref, w_ref, b_ref, o_ref):
        a = a_ref[0] + a_ref[1]
        node = jnp.dot(a, w_ref[...], preferred_element_type=jnp.float32)
        gids = lax.broadcasted_iota(jnp.int32, (NUM_GRAPHS, N_NODES), 0)
        onehot = (b_ref[...] == gids).astype(jnp.float32)
        pooled = jnp.dot(onehot, node, preferred_element_type=jnp.float32)
        counts = jnp.sum(onehot, axis=1, keepdims=True)
        o_ref[...] = pooled / jnp.maximum(counts, 1.0)

    return pl.pallas_call(
        body,
        out_shape=jax.ShapeDtypeStruct((NUM_GRAPHS, HIDDEN), jnp.float32),
    )(A, wf, batch2d)


def _pad2(w, rp, cp):
    return jnp.zeros((rp, cp), w.dtype).at[: w.shape[0], : w.shape[1]].set(w)


def _pad1(v, n):
    return jnp.zeros((n,), v.dtype).at[: v.shape[0]].set(v)


def kernel(x, edge_index, edge_attr, batch, W_edge_init, W_conv, b_conv,
           g_conv, be_conv, W_e2n, b_e2n, g_e2n, be_e2n, W_ffn):
    row = edge_index[0]
    col = edge_index[1]
    W1p = _pad2(W_edge_init[:NODE_DIM], NODE_DIM, HP)
    W2p = _pad2(W_edge_init[NODE_DIM:], EDGE_DIM, HP)
    Wcp = _pad2(W_conv, HP, HP)
    bcp = _pad1(b_conv, HP).reshape(1, HP)
    gcp = _pad1(g_conv, HP).reshape(1, HP)
    bep = _pad1(be_conv, HP).reshape(1, HP)
    Wfp = _pad2(W_ffn, HP, HIDDEN)
    zerosA = jnp.zeros((N_NODES, HP), jnp.float32)
    ones = jnp.ones((1, HP), jnp.float32)
    zeros1 = jnp.zeros((1, HP), jnp.float32)

    X1 = _tc_matmul(x, W1p, 1000)
    T = _tc_matmul(edge_attr, W2p, 2000)
    (Z,) = _sc_gather_init(X1, T, row)

    isc, ish = ones, zeros1
    stats = None
    for l in range(DEPTH):
        if l == 0:
            A = _sc_scatter_init(Z, col, zerosA, ones.reshape(HP),
                                 zeros1.reshape(HP))
            Q = _tc_q_init(Z, Wcp, ones, zeros1)
        else:
            isc, ish = _tc_stats(stats, gcp, bep)
            A = _sc_scatter_norm(Z, col, zerosA, isc.reshape(HP),
                                 ish.reshape(HP))
            Q = _tc_q_norm(Z, Wcp, isc, ish)
        Pp = _tc_pprime(A, Wcp, bcp)
        Z, stats = _sc_gather_conv(Pp, Q, row)

    isc, ish = _tc_stats(stats, gcp, bep)
    A = _sc_scatter_norm(Z, col, zerosA, isc.reshape(HP), ish.reshape(HP))
    return _tc_final(A, Wfp, batch.reshape(1, N_NODES))


# speedup vs baseline: 2.3524x; 2.3524x over previous
import functools

import jax
import jax.numpy as jnp
from jax import lax
from jax.experimental import pallas as pl
from jax.experimental.pallas import tpu as pltpu
from jax.experimental.pallas import tpu_sc as plsc

N_NODES = 10000
N_EDGES = 320000
NODE_DIM = 128
EDGE_DIM = 16
HIDDEN = 200
HP = 256
NSL = HP // 16
HH = 128
NSL_H = HH // 16
DEPTH = 4
NUM_GRAPHS = 64
EPS = 1e-5

NC = 2
NSUB = 16
NW = NC * NSUB
CHUNK = 80
E_PER_W = N_EDGES // NW
NCHUNK_G = E_PER_W // CHUNK
E_PER_T = N_EDGES // NSUB
NCHUNK_S = E_PER_T // CHUNK
ROWS_T = 624
ROWS_REM = N_NODES - ROWS_T * NSUB

_mesh = plsc.VectorSubcoreMesh(core_axis_name="c", subcore_axis_name="s")


def _make_sc_scatter(use_norm):
    @functools.partial(
        pl.kernel,
        mesh=_mesh,
        out_type=jax.ShapeDtypeStruct((N_NODES, HP), jnp.float32),
        scratch_types=[
            pltpu.VMEM_SHARED((N_NODES, HH), jnp.float32),
            pltpu.VMEM((CHUNK, HH), jnp.float32),
            pltpu.VMEM((CHUNK,), jnp.int32),
            pltpu.VMEM((HP,), jnp.float32),
            pltpu.VMEM((HP,), jnp.float32),
        ],
    )
    def k(z_hbm, col_hbm, zeros_hbm, isc_hbm, ish_hbm, out_hbm,
          acc_sh, z_v, idx_v, isc_v, ish_v):
        cid = lax.axis_index("c")
        sid = lax.axis_index("s")
        coff = pl.multiple_of(cid * HH, HH)
        roff = pl.multiple_of(sid * ROWS_T, 8)
        pltpu.sync_copy(zeros_hbm.at[pl.ds(roff, ROWS_T)],
                        acc_sh.at[pl.ds(roff, ROWS_T)])

        @pl.when(sid == NSUB - 1)
        def _():
            pltpu.sync_copy(zeros_hbm.at[pl.ds(ROWS_T * NSUB, ROWS_REM)],
                            acc_sh.at[pl.ds(ROWS_T * NSUB, ROWS_REM)])

        plsc.subcore_barrier()
        pltpu.sync_copy(isc_hbm, isc_v)
        pltpu.sync_copy(ish_hbm, ish_v)
        iscs = [isc_v[pl.ds(pl.multiple_of(coff + s * 16, 16), 16)]
                for s in range(NSL_H)]
        ishs = [ish_v[pl.ds(pl.multiple_of(coff + s * 16, 16), 16)]
                for s in range(NSL_H)]
        base = sid * E_PER_T

        def chunk_body(i, carry):
            off = pl.multiple_of(base + i * CHUNK, 8)
            pltpu.sync_copy(z_hbm.at[pl.ds(off, CHUNK), pl.ds(coff, HH)], z_v)
            pltpu.sync_copy(col_hbm.at[pl.ds(off, CHUNK)], idx_v)

            def row_body(r, c2):
                for s in range(NSL_H):
                    sl = pl.ds(s * 16, 16)
                    z = z_v[r, sl]
                    if use_norm:
                        t = z * iscs[s] + ishs[s]
                        h = jnp.maximum(t, 0.0)
                        h = h + h
                    else:
                        h = jnp.maximum(z, 0.01 * z)
                    z_v[r, sl] = h
                return c2

            lax.fori_loop(0, CHUNK, row_body, 0)
            pltpu.sync_copy(z_v, acc_sh.at[idx_v], add=True)
            return carry

        lax.fori_loop(0, NCHUNK_S, chunk_body, 0)
        plsc.subcore_barrier()
        pltpu.sync_copy(acc_sh.at[pl.ds(roff, ROWS_T)],
                        out_hbm.at[pl.ds(roff, ROWS_T), pl.ds(coff, HH)])

        @pl.when(sid == NSUB - 1)
        def _():
            pltpu.sync_copy(acc_sh.at[pl.ds(ROWS_T * NSUB, ROWS_REM)],
                            out_hbm.at[pl.ds(ROWS_T * NSUB, ROWS_REM),
                                       pl.ds(coff, HH)])

    return k


def _make_sc_gather(flip, do_stats):
    out_type = [jax.ShapeDtypeStruct((N_EDGES, HP), jnp.float32)]
    if do_stats:
        out_type.append(jax.ShapeDtypeStruct((NW, 2, HP), jnp.float32))

    @functools.partial(
        pl.kernel,
        mesh=_mesh,
        out_type=out_type,
        scratch_types=[
            pltpu.VMEM((CHUNK, HP), jnp.float32),
            pltpu.VMEM((CHUNK, HP), jnp.float32),
            pltpu.VMEM((CHUNK,), jnp.int32),
            pltpu.VMEM((2, HP), jnp.float32),
            pltpu.SemaphoreType.DMA,
        ],
    )
    def k(table_hbm, q_hbm, idx_hbm, *refs):
        if do_stats:
            out_hbm, stats_hbm, q_v, g_v, idx_v, st_v, sem = refs
        else:
            out_hbm, q_v, g_v, idx_v, st_v, sem = refs
        cid = lax.axis_index("c")
        sid = lax.axis_index("s")
        wid = cid * NSUB + sid
        base = wid * E_PER_W
        zero = jnp.zeros((16,), jnp.float32)
        init = tuple([zero] * NSL) + tuple([zero] * NSL)

        def chunk_body(i, carry):
            off = pl.multiple_of(base + i * CHUNK, 8)
            pltpu.sync_copy(idx_hbm.at[pl.ds(off, CHUNK)], idx_v)
            pltpu.sync_copy(q_hbm.at[pl.ds(off, CHUNK)], q_v)
            pltpu.async_copy(table_hbm.at[idx_v], g_v, sem).wait()

            def pair_body(r, c2):
                sums = c2[:NSL]
                sqs = c2[NSL:]
                new_s = []
                new_q = []
                for s in range(NSL):
                    sl = pl.ds(s * 16, 16)
                    qa = q_v[2 * r, sl]
                    qb = q_v[2 * r + 1, sl]
                    ga = g_v[2 * r, sl]
                    gb = g_v[2 * r + 1, sl]
                    if flip:
                        y0 = ga - qb
                        y1 = gb - qa
                    else:
                        y0 = ga + qa
                        y1 = gb + qb
                    q_v[2 * r, sl] = y0
                    q_v[2 * r + 1, sl] = y1
                    if do_stats:
                        new_s.append(sums[s] + (y0 + y1))
                        new_q.append(sqs[s] + (y0 * y0 + y1 * y1))
                if do_stats:
                    return tuple(new_s) + tuple(new_q)
                return c2

            carry = lax.fori_loop(0, CHUNK // 2, pair_body, carry)
            pltpu.sync_copy(q_v, out_hbm.at[pl.ds(off, CHUNK)])
            return carry

        fin = lax.fori_loop(0, NCHUNK_G, chunk_body, init)
        if do_stats:
            for s in range(NSL):
                st_v[0, pl.ds(s * 16, 16)] = fin[s]
                st_v[1, pl.ds(s * 16, 16)] = fin[NSL + s]
            pltpu.sync_copy(st_v, stats_hbm.at[wid])

    return k


_sc_scatter_init = _make_sc_scatter(use_norm=False)
_sc_scatter_norm = _make_sc_scatter(use_norm=True)
_sc_gather_init = _make_sc_gather(flip=False, do_stats=False)
_sc_gather_conv = _make_sc_gather(flip=True, do_stats=True)


def _mm_kernel(x_ref, w_ref, o_ref):
    o_ref[...] = jnp.dot(x_ref[...], w_ref[...],
                         preferred_element_type=jnp.float32)


def _tc_matmul(x, w, row_block):
    m, kdim = x.shape
    n = w.shape[1]
    return pl.pallas_call(
        _mm_kernel,
        grid=(m // row_block,),
        in_specs=[
            pl.BlockSpec((row_block, kdim), lambda i: (i, 0)),
            pl.BlockSpec((kdim, n), lambda i: (0, 0)),
        ],
        out_specs=pl.BlockSpec((row_block, n), lambda i: (i, 0)),
        out_shape=jax.ShapeDtypeStruct((m, n), jnp.float32),
    )(x, w)


def _make_tc_act_matmul(use_norm, row_block):
    def body(z_ref, w_ref, isc_ref, ish_ref, o_ref):
        z = z_ref[...]
        if use_norm:
            t = z * isc_ref[...] + ish_ref[...]
            h = jnp.maximum(t, 0.0)
            h = h + h
        else:
            h = jnp.maximum(z, 0.01 * z)
        o_ref[...] = jnp.dot(h, w_ref[...], preferred_element_type=jnp.float32)

    def call(z, w, isc, ish):
        m, kdim = z.shape
        n = w.shape[1]
        return pl.pallas_call(
            body,
            grid=(m // row_block,),
            in_specs=[
                pl.BlockSpec((row_block, kdim), lambda i: (i, 0)),
                pl.BlockSpec((kdim, n), lambda i: (0, 0)),
                pl.BlockSpec((1, kdim), lambda i: (0, 0)),
                pl.BlockSpec((1, kdim), lambda i: (0, 0)),
            ],
            out_specs=pl.BlockSpec((row_block, n), lambda i: (i, 0)),
            out_shape=jax.ShapeDtypeStruct((m, n), jnp.float32),
        )(z, w, isc, ish)

    return call


_tc_q_init = _make_tc_act_matmul(use_norm=False, row_block=640)
_tc_q_norm = _make_tc_act_matmul(use_norm=True, row_block=640)


def _tc_pprime(A, w, b):
    def body(a_ref, w_ref, b_ref, o_ref):
        o_ref[...] = jnp.dot(a_ref[...], w_ref[...],
                             preferred_element_type=jnp.float32) + b_ref[...]

    return pl.pallas_call(
        body,
        out_shape=jax.ShapeDtypeStruct((N_NODES, HP), jnp.float32),
    )(A, w, b)


def _tc_stats(stats, gcp, bep):
    def body(st_ref, g_ref, be_ref, isc_ref, ish_ref):
        s = jnp.sum(st_ref[:, 0, :], axis=0, keepdims=True)
        sq = jnp.sum(st_ref[:, 1, :], axis=0, keepdims=True)
        mu = s / N_EDGES
        var = sq / N_EDGES - mu * mu
        isc = g_ref[...] * jax.lax.rsqrt(var + EPS)
        isc_ref[...] = isc
        ish_ref[...] = be_ref[...] - mu * isc

    return pl.pallas_call(
        body,
        out_shape=[
            jax.ShapeDtypeStruct((1, HP), jnp.float32),
            jax.ShapeDtypeStruct((1, HP), jnp.float32),
        ],
    )(stats, gcp, bep)


def _tc_final(A, wf, batch2d):
    def body(a_ref, w_ref, b_ref, o_ref):
        node = jnp.dot(a_ref[...], w_ref[...],
                       preferred_element_type=jnp.float32)
        gids = lax.broadcasted_iota(jnp.int32, (NUM_GRAPHS, N_NODES), 0)
        onehot = (b_ref[...] == gids).astype(jnp.float32)
        pooled = jnp.dot(onehot, node, preferred_element_type=jnp.float32)
        counts = jnp.sum(onehot, axis=1, keepdims=True)
        o_ref[...] = pooled / jnp.maximum(counts, 1.0)

    return pl.pallas_call(
        body,
        out_shape=jax.ShapeDtypeStruct((NUM_GRAPHS, HIDDEN), jnp.float32),
    )(A, wf, batch2d)


def _pad2(w, rp, cp):
    return jnp.zeros((rp, cp), w.dtype).at[: w.shape[0], : w.shape[1]].set(w)


def _pad1(v, n):
    return jnp.zeros((n,), v.dtype).at[: v.shape[0]].set(v)


def kernel(x, edge_index, edge_attr, batch, W_edge_init, W_conv, b_conv,
           g_conv, be_conv, W_e2n, b_e2n, g_e2n, be_e2n, W_ffn):
    row = edge_index[0]
    col = edge_index[1]
    W1p = _pad2(W_edge_init[:NODE_DIM], NODE_DIM, HP)
    W2p = _pad2(W_edge_init[NODE_DIM:], EDGE_DIM, HP)
    Wcp = _pad2(W_conv, HP, HP)
    bcp = _pad1(b_conv, HP).reshape(1, HP)
    gcp = _pad1(g_conv, HP).reshape(1, HP)
    bep = _pad1(be_conv, HP).reshape(1, HP)
    Wfp = _pad2(W_ffn, HP, HIDDEN)
    zerosA = jnp.zeros((N_NODES, HH), jnp.float32)
    ones = jnp.ones((1, HP), jnp.float32)
    zeros1 = jnp.zeros((1, HP), jnp.float32)

    X1 = _tc_matmul(x, W1p, 1000)
    T = _tc_matmul(edge_attr, W2p, 2000)
    (Z,) = _sc_gather_init(X1, T, row)

    isc, ish = ones, zeros1
    stats = None
    for l in range(DEPTH):
        if l == 0:
            A = _sc_scatter_init(Z, col, zerosA, ones.reshape(HP),
                                 zeros1.reshape(HP))
            Q = _tc_q_init(Z, Wcp, ones, zeros1)
        else:
            isc, ish = _tc_stats(stats, gcp, bep)
            A = _sc_scatter_norm(Z, col, zerosA, isc.reshape(HP),
                                 ish.reshape(HP))
            Q = _tc_q_norm(Z, Wcp, isc, ish)
        Pp = _tc_pprime(A, Wcp, bcp)
        Z, stats = _sc_gather_conv(Pp, Q, row)

    isc, ish = _tc_stats(stats, gcp, bep)
    A = _sc_scatter_norm(Z, col, zerosA, isc.reshape(HP), ish.reshape(HP))
    return _tc_final(A, Wfp, batch.reshape(1, N_NODES))


# speedup vs baseline: 4.1385x; 1.7593x over previous
import functools

import jax
import jax.numpy as jnp
from jax import lax
from jax.experimental import pallas as pl
from jax.experimental.pallas import tpu as pltpu
from jax.experimental.pallas import tpu_sc as plsc

N_NODES = 10000
N_EDGES = 320000
NODE_DIM = 128
EDGE_DIM = 16
HIDDEN = 200
HP = 256
NSL = HP // 16
HH = 128
NSL_H = HH // 16
DEPTH = 4
NUM_GRAPHS = 64
EPS = 1e-5

NC = 2
NSUB = 16
NW = NC * NSUB
NBUF = 5
CH_G = 40
E_PER_W = N_EDGES // NW
NCH_G = E_PER_W // CH_G
CH_S = 40
E_PER_T = N_EDGES // NSUB
NCH_S = E_PER_T // CH_S
ROWS_T = 624
ROWS_REM = N_NODES - ROWS_T * NSUB

_mesh = plsc.VectorSubcoreMesh(core_axis_name="c", subcore_axis_name="s")


def _make_sc_scatter(use_norm):
    @functools.partial(
        pl.kernel,
        mesh=_mesh,
        out_type=jax.ShapeDtypeStruct((N_NODES, HP), jnp.float32),
        scratch_types=(
            [pltpu.VMEM_SHARED((N_NODES, HH), jnp.float32)]
            + [pltpu.VMEM((CH_S, HH), jnp.float32) for _ in range(NBUF)]
            + [pltpu.VMEM((CH_S,), jnp.int32) for _ in range(NBUF)]
            + [
                pltpu.VMEM((HP,), jnp.float32),
                pltpu.VMEM((HP,), jnp.float32),
                pltpu.SemaphoreType.DMA((NBUF,)),
                pltpu.SemaphoreType.DMA((NBUF,)),
                pltpu.SemaphoreType.DMA((NBUF,)),
            ]
        ),
    )
    def k(z_hbm, col_hbm, zeros_hbm, isc_hbm, ish_hbm, out_hbm, acc_sh, *refs):
        z_v = refs[:NBUF]
        idx_v = refs[NBUF:2 * NBUF]
        isc_v, ish_v, sz, si, sc = refs[2 * NBUF:]
        cid = lax.axis_index("c")
        sid = lax.axis_index("s")
        coff = pl.multiple_of(cid * HH, HH)
        roff = pl.multiple_of(sid * ROWS_T, 8)
        pltpu.sync_copy(zeros_hbm.at[pl.ds(roff, ROWS_T)],
                        acc_sh.at[pl.ds(roff, ROWS_T)])

        @pl.when(sid == NSUB - 1)
        def _():
            pltpu.sync_copy(zeros_hbm.at[pl.ds(ROWS_T * NSUB, ROWS_REM)],
                            acc_sh.at[pl.ds(ROWS_T * NSUB, ROWS_REM)])

        plsc.subcore_barrier()
        pltpu.sync_copy(isc_hbm, isc_v)
        pltpu.sync_copy(ish_hbm, ish_v)
        iscs = [isc_v[pl.ds(pl.multiple_of(coff + s * 16, 16), 16)]
                for s in range(NSL_H)]
        ishs = [ish_v[pl.ds(pl.multiple_of(coff + s * 16, 16), 16)]
                for s in range(NSL_H)]
        base = sid * E_PER_T

        def z_copy(c, b):
            off = pl.multiple_of(base + c * CH_S, 8)
            return pltpu.make_async_copy(
                z_hbm.at[pl.ds(off, CH_S), pl.ds(coff, HH)], z_v[b], sz.at[b])

        def i_copy(c, b):
            off = pl.multiple_of(base + c * CH_S, 8)
            return pltpu.make_async_copy(
                col_hbm.at[pl.ds(off, CH_S)], idx_v[b], si.at[b])

        def a_copy(b):
            return pltpu.make_async_copy(
                z_v[b], acc_sh.at[idx_v[b]], sc.at[b])

        for c in range(2):
            z_copy(c, c).start()
            i_copy(c, c).start()

        def group_body(c5, carry):
            for kk in range(NBUF):
                c = c5 * NBUF + kk
                bn = (kk + 2) % NBUF

                @pl.when(jnp.logical_and(c >= 3, c + 2 < NCH_S))
                def _():
                    a_copy(bn).wait()

                @pl.when(c + 2 < NCH_S)
                def _():
                    z_copy(c + 2, bn).start()
                    i_copy(c + 2, bn).start()

                z_copy(c, kk).wait()
                i_copy(c, kk).wait()

                def row_body(r, c2):
                    for s in range(NSL_H):
                        sl = pl.ds(s * 16, 16)
                        z = z_v[kk][r, sl]
                        if use_norm:
                            t = z * iscs[s] + ishs[s]
                            h = jnp.maximum(t, 0.0)
                            h = h + h
                        else:
                            h = jnp.maximum(z, 0.01 * z)
                        z_v[kk][r, sl] = h
                    return c2

                lax.fori_loop(0, CH_S, row_body, 0)
                a_copy(kk).start(add=True)
            return carry

        lax.fori_loop(0, NCH_S // NBUF, group_body, 0)
        for b in range(NBUF):
            a_copy(b).wait()
        plsc.subcore_barrier()
        pltpu.sync_copy(acc_sh.at[pl.ds(roff, ROWS_T)],
                        out_hbm.at[pl.ds(roff, ROWS_T), pl.ds(coff, HH)])

        @pl.when(sid == NSUB - 1)
        def _():
            pltpu.sync_copy(acc_sh.at[pl.ds(ROWS_T * NSUB, ROWS_REM)],
                            out_hbm.at[pl.ds(ROWS_T * NSUB, ROWS_REM),
                                       pl.ds(coff, HH)])

    return k


def _make_sc_gather(flip, do_stats):
    out_type = [jax.ShapeDtypeStruct((N_EDGES, HP), jnp.float32)]
    if do_stats:
        out_type.append(jax.ShapeDtypeStruct((NW, 2, HP), jnp.float32))

    @functools.partial(
        pl.kernel,
        mesh=_mesh,
        out_type=out_type,
        scratch_types=(
            [pltpu.VMEM((CH_G, HP), jnp.float32) for _ in range(NBUF)]
            + [pltpu.VMEM((CH_G, HP), jnp.float32) for _ in range(NBUF)]
            + [pltpu.VMEM((CH_G,), jnp.int32) for _ in range(NBUF)]
            + [
                pltpu.VMEM((2, HP), jnp.float32),
                pltpu.SemaphoreType.DMA((NBUF,)),
                pltpu.SemaphoreType.DMA((NBUF,)),
                pltpu.SemaphoreType.DMA((NBUF,)),
                pltpu.SemaphoreType.DMA((NBUF,)),
            ]
        ),
    )
    def k(table_hbm, q_hbm, idx_hbm, *refs):
        if do_stats:
            out_hbm, stats_hbm = refs[0], refs[1]
            refs = refs[2:]
        else:
            out_hbm = refs[0]
            stats_hbm = None
            refs = refs[1:]
        q_v = refs[:NBUF]
        g_v = refs[NBUF:2 * NBUF]
        idx_v = refs[2 * NBUF:3 * NBUF]
        st_v, sq, sg, si, so = refs[3 * NBUF:]
        cid = lax.axis_index("c")
        sid = lax.axis_index("s")
        wid = cid * NSUB + sid
        base = wid * E_PER_W
        zero = jnp.zeros((16,), jnp.float32)
        init = tuple([zero] * NSL) + tuple([zero] * NSL)

        def q_copy(c, b):
            off = pl.multiple_of(base + c * CH_G, 8)
            return pltpu.make_async_copy(
                q_hbm.at[pl.ds(off, CH_G)], q_v[b], sq.at[b])

        def i_copy(c, b):
            off = pl.multiple_of(base + c * CH_G, 8)
            return pltpu.make_async_copy(
                idx_hbm.at[pl.ds(off, CH_G)], idx_v[b], si.at[b])

        def g_copy(b):
            return pltpu.make_async_copy(table_hbm.at[idx_v[b]], g_v[b],
                                         sg.at[b])

        def o_copy(c, b):
            off = pl.multiple_of(base + c * CH_G, 8)
            return pltpu.make_async_copy(
                q_v[b], out_hbm.at[pl.ds(off, CH_G)], so.at[b])

        for c in range(2):
            i_copy(c, c).start()
            q_copy(c, c).start()
        i_copy(0, 0).wait()
        g_copy(0).start()

        def group_body(c5, carry):
            for kk in range(NBUF):
                c = c5 * NBUF + kk
                bn = (kk + 2) % NBUF
                b1 = (kk + 1) % NBUF

                @pl.when(jnp.logical_and(c >= 3, c + 2 < NCH_G))
                def _():
                    o_copy(c, bn).wait()

                @pl.when(c + 2 < NCH_G)
                def _():
                    q_copy(c + 2, bn).start()
                    i_copy(c + 2, bn).start()

                @pl.when(c + 1 < NCH_G)
                def _():
                    i_copy(c + 1, b1).wait()
                    g_copy(b1).start()

                q_copy(c, kk).wait()
                g_copy(kk).wait()

                def pair_body(r, c2):
                    sums = c2[:NSL]
                    sqs = c2[NSL:]
                    new_s = []
                    new_q = []
                    for s in range(NSL):
                        sl = pl.ds(s * 16, 16)
                        qa = q_v[kk][2 * r, sl]
                        qb = q_v[kk][2 * r + 1, sl]
                        ga = g_v[kk][2 * r, sl]
                        gb = g_v[kk][2 * r + 1, sl]
                        if flip:
                            y0 = ga - qb
                            y1 = gb - qa
                        else:
                            y0 = ga + qa
                            y1 = gb + qb
                        q_v[kk][2 * r, sl] = y0
                        q_v[kk][2 * r + 1, sl] = y1
                        if do_stats:
                            new_s.append(sums[s] + (y0 + y1))
                            new_q.append(sqs[s] + (y0 * y0 + y1 * y1))
                    if do_stats:
                        return tuple(new_s) + tuple(new_q)
                    return c2

                carry = lax.fori_loop(0, CH_G // 2, pair_body, carry)
                o_copy(c, kk).start()
            return carry

        fin = lax.fori_loop(0, NCH_G // NBUF, group_body, init)
        for b in range(NBUF):
            o_copy(0, b).wait()
        if do_stats:
            for s in range(NSL):
                st_v[0, pl.ds(s * 16, 16)] = fin[s]
                st_v[1, pl.ds(s * 16, 16)] = fin[NSL + s]
            pltpu.sync_copy(st_v, stats_hbm.at[wid])

    return k


_sc_scatter_init = _make_sc_scatter(use_norm=False)
_sc_scatter_norm = _make_sc_scatter(use_norm=True)
_sc_gather_init = _make_sc_gather(flip=False, do_stats=False)
_sc_gather_conv = _make_sc_gather(flip=True, do_stats=True)


def _mm_kernel(x_ref, w_ref, o_ref):
    o_ref[...] = jnp.dot(x_ref[...], w_ref[...],
                         preferred_element_type=jnp.float32)


def _tc_matmul(x, w, row_block):
    m, kdim = x.shape
    n = w.shape[1]
    return pl.pallas_call(
        _mm_kernel,
        grid=(m // row_block,),
        in_specs=[
            pl.BlockSpec((row_block, kdim), lambda i: (i, 0)),
            pl.BlockSpec((kdim, n), lambda i: (0, 0)),
        ],
        out_specs=pl.BlockSpec((row_block, n), lambda i: (i, 0)),
        out_shape=jax.ShapeDtypeStruct((m, n), jnp.float32),
    )(x, w)


def _make_tc_act_matmul(use_norm, row_block):
    def body(z_ref, w_ref, isc_ref, ish_ref, o_ref):
        z = z_ref[...]
        if use_norm:
            t = z * isc_ref[...] + ish_ref[...]
            h = jnp.maximum(t, 0.0)
            h = h + h
        else:
            h = jnp.maximum(z, 0.01 * z)
        o_ref[...] = jnp.dot(h, w_ref[...], preferred_element_type=jnp.float32)

    def call(z, w, isc, ish):
        m, kdim = z.shape
        n = w.shape[1]
        return pl.pallas_call(
            body,
            grid=(m // row_block,),
            in_specs=[
                pl.BlockSpec((row_block, kdim), lambda i: (i, 0)),
                pl.BlockSpec((kdim, n), lambda i: (0, 0)),
                pl.BlockSpec((1, kdim), lambda i: (0, 0)),
                pl.BlockSpec((1, kdim), lambda i: (0, 0)),
            ],
            out_specs=pl.BlockSpec((row_block, n), lambda i: (i, 0)),
            out_shape=jax.ShapeDtypeStruct((m, n), jnp.float32),
        )(z, w, isc, ish)

    return call


_tc_q_init = _make_tc_act_matmul(use_norm=False, row_block=640)
_tc_q_norm = _make_tc_act_matmul(use_norm=True, row_block=640)


def _tc_pprime(A, w, b):
    def body(a_ref, w_ref, b_ref, o_ref):
        o_ref[...] = jnp.dot(a_ref[...], w_ref[...],
                             preferred_element_type=jnp.float32) + b_ref[...]

    return pl.pallas_call(
        body,
        out_shape=jax.ShapeDtypeStruct((N_NODES, HP), jnp.float32),
    )(A, w, b)


def _tc_stats(stats, gcp, bep):
    def body(st_ref, g_ref, be_ref, isc_ref, ish_ref):
        s = jnp.sum(st_ref[:, 0, :], axis=0, keepdims=True)
        sq = jnp.sum(st_ref[:, 1, :], axis=0, keepdims=True)
        mu = s / N_EDGES
        var = sq / N_EDGES - mu * mu
        isc = g_ref[...] * jax.lax.rsqrt(var + EPS)
        isc_ref[...] = isc
        ish_ref[...] = be_ref[...] - mu * isc

    return pl.pallas_call(
        body,
        out_shape=[
            jax.ShapeDtypeStruct((1, HP), jnp.float32),
            jax.ShapeDtypeStruct((1, HP), jnp.float32),
        ],
    )(stats, gcp, bep)


def _tc_final(A, wf, batch2d):
    def body(a_ref, w_ref, b_ref, o_ref):
        node = jnp.dot(a_ref[...], w_ref[...],
                       preferred_element_type=jnp.float32)
        gids = lax.broadcasted_iota(jnp.int32, (NUM_GRAPHS, N_NODES), 0)
        onehot = (b_ref[...] == gids).astype(jnp.float32)
        pooled = jnp.dot(onehot, node, preferred_element_type=jnp.float32)
        counts = jnp.sum(onehot, axis=1, keepdims=True)
        o_ref[...] = pooled / jnp.maximum(counts, 1.0)

    return pl.pallas_call(
        body,
        out_shape=jax.ShapeDtypeStruct((NUM_GRAPHS, HIDDEN), jnp.float32),
    )(A, wf, batch2d)


def _pad2(w, rp, cp):
    return jnp.zeros((rp, cp), w.dtype).at[: w.shape[0], : w.shape[1]].set(w)


def _pad1(v, n):
    return jnp.zeros((n,), v.dtype).at[: v.shape[0]].set(v)


def kernel(x, edge_index, edge_attr, batch, W_edge_init, W_conv, b_conv,
           g_conv, be_conv, W_e2n, b_e2n, g_e2n, be_e2n, W_ffn):
    row = edge_index[0]
    col = edge_index[1]
    W1p = _pad2(W_edge_init[:NODE_DIM], NODE_DIM, HP)
    W2p = _pad2(W_edge_init[NODE_DIM:], EDGE_DIM, HP)
    Wcp = _pad2(W_conv, HP, HP)
    bcp = _pad1(b_conv, HP).reshape(1, HP)
    gcp = _pad1(g_conv, HP).reshape(1, HP)
    bep = _pad1(be_conv, HP).reshape(1, HP)
    Wfp = _pad2(W_ffn, HP, HIDDEN)
    zerosA = jnp.zeros((N_NODES, HH), jnp.float32)
    ones = jnp.ones((1, HP), jnp.float32)
    zeros1 = jnp.zeros((1, HP), jnp.float32)

    X1 = _tc_matmul(x, W1p, 1000)
    T = _tc_matmul(edge_attr, W2p, 2000)
    (Z,) = _sc_gather_init(X1, T, row)

    isc, ish = ones, zeros1
    stats = None
    for l in range(DEPTH):
        if l == 0:
            A = _sc_scatter_init(Z, col, zerosA, ones.reshape(HP),
                                 zeros1.reshape(HP))
            Q = _tc_q_init(Z, Wcp, ones, zeros1)
        else:
            isc, ish = _tc_stats(stats, gcp, bep)
            A = _sc_scatter_norm(Z, col, zerosA, isc.reshape(HP),
                                 ish.reshape(HP))
            Q = _tc_q_norm(Z, Wcp, isc, ish)
        Pp = _tc_pprime(A, Wcp, bcp)
        Z, stats = _sc_gather_conv(Pp, Q, row)

    isc, ish = _tc_stats(stats, gcp, bep)
    A = _sc_scatter_norm(Z, col, zerosA, isc.reshape(HP), ish.reshape(HP))
    return _tc_final(A, Wfp, batch.reshape(1, N_NODES))
